# baseline (speedup 1.0000x reference)
import functools

import numpy as np
import jax
import jax.numpy as jnp
from jax import lax
from jax.experimental import pallas as pl
from jax.experimental.pallas import tpu as pltpu
from jax.experimental.pallas import tpu_sc as plsc

VOCAB = 1000000
D = 64
B = 1024
S = 200

NC = 2
NS = 16
NW = NC * NS
TOTAL = B * S
PER_W = TOTAL // NW
CHUNK = 160
NCHUNK = PER_W // CHUNK
NPAIR = NCHUNK // 2
SLICES = [(0, 128), (128, CHUNK - 128)] if CHUNK > 128 else [(0, CHUNK)]


def _make_pe():
    pos = np.arange(S, dtype=np.float32)[:, None]
    div = np.exp(np.arange(0, D, 2, dtype=np.float32) * -(np.log(10000.0) / D))
    pe = np.zeros((S, D), dtype=np.float32)
    pe[:, 0::2] = np.sin(pos * div)
    pe[:, 1::2] = np.cos(pos * div)
    return pe


_PE = _make_pe()


def _comb_body(pe_ref, seg_ref, out_ref):
    pe = pe_ref[...]
    for l in range(3):
        out_ref[:, l * D:(l + 1) * D] = pe + seg_ref[l, :][None, :]


def _build_comb(seg_table):
    comb2 = pl.pallas_call(
        _comb_body,
        out_shape=jax.ShapeDtypeStruct((S, 3 * D), jnp.float32),
    )(_PE, seg_table)
    return comb2.reshape(3 * S, D)


_mesh = plsc.VectorSubcoreMesh(core_axis_name="c", subcore_axis_name="s")


@functools.partial(
    pl.kernel,
    mesh=_mesh,
    out_type=jax.ShapeDtypeStruct((TOTAL, D), jnp.float32),
    scratch_types=[
        pltpu.VMEM((CHUNK,), jnp.int32),
        pltpu.VMEM((CHUNK,), jnp.int32),
        pltpu.VMEM((CHUNK, D), jnp.float32),
        pltpu.VMEM((CHUNK, D), jnp.float32),
        pltpu.VMEM((CHUNK, D), jnp.float32),
        pltpu.VMEM((CHUNK,), jnp.int32),
        pltpu.VMEM((CHUNK,), jnp.int32),
        pltpu.VMEM((CHUNK, D), jnp.float32),
        pltpu.VMEM((CHUNK, D), jnp.float32),
        pltpu.VMEM((CHUNK, D), jnp.float32),
        pltpu.SemaphoreType.DMA,
        pltpu.SemaphoreType.DMA,
        pltpu.SemaphoreType.DMA,
        pltpu.SemaphoreType.DMA,
    ],
    compiler_params=pltpu.CompilerParams(use_tc_tiling_on_sc=False),
)
def _emb(seq_hbm, lab_hbm, tok_hbm, comb_hbm, out_hbm,
         tok_idx_a, cmb_idx_a, tok_va, cmb_va, out_va,
         tok_idx_b, cmb_idx_b, tok_vb, cmb_vb, out_vb,
         sem_ga, sem_gb, sem_oa, sem_ob):
    wid = lax.axis_index("s") * NC + lax.axis_index("c")
    w0 = wid * PER_W

    def prep_and_fire(c, tok_idx, cmb_idx, tok_v, cmb_v, sem_g):
        base = pl.multiple_of(w0 + c * CHUNK, CHUNK)
        pltpu.sync_copy(seq_hbm.at[pl.ds(base, CHUNK)], tok_idx)
        pltpu.sync_copy(lab_hbm.at[pl.ds(base, CHUNK)], cmb_idx)
        for k in range(CHUNK // 16):
            sl = pl.ds(k * 16, 16)
            lab = cmb_idx[sl]
            v = base + k * 16 + lax.iota(jnp.int32, 16)
            cmb_idx[sl] = lax.rem(v, S) * 3 + lab
        for off, ln in SLICES:
            pltpu.async_copy(tok_hbm.at[tok_idx.at[pl.ds(off, ln)]],
                             tok_v.at[pl.ds(off, ln)], sem_g)
            pltpu.async_copy(comb_hbm.at[cmb_idx.at[pl.ds(off, ln)]],
                             cmb_v.at[pl.ds(off, ln)], sem_g)

    def wait_gathers(tok_v, cmb_v, sem_g):
        pltpu.make_async_copy(tok_hbm.at[pl.ds(0, CHUNK)], tok_v, sem_g).wait()
        pltpu.make_async_copy(tok_hbm.at[pl.ds(0, CHUNK)], cmb_v, sem_g).wait()

    def add_chunk(tok_v, cmb_v, out_v):
        def add_rows(r, carry2):
            for j in range(2):
                i = r * 2 + j
                for k in range(D // 16):
                    sl = pl.ds(k * 16, 16)
                    out_v[i, sl] = tok_v[i, sl] + cmb_v[i, sl]
            return carry2

        lax.fori_loop(0, CHUNK // 2, add_rows, 0)

    def fire_writeback(c, out_v, sem_o):
        base = pl.multiple_of(w0 + c * CHUNK, CHUNK)
        pltpu.async_copy(out_v, out_hbm.at[pl.ds(base, CHUNK)], sem_o)

    def wait_writeback(out_v, sem_o):
        pltpu.make_async_copy(out_v, out_hbm.at[pl.ds(0, CHUNK)], sem_o).wait()

    prep_and_fire(0, tok_idx_a, cmb_idx_a, tok_va, cmb_va, sem_ga)
    prep_and_fire(1, tok_idx_b, cmb_idx_b, tok_vb, cmb_vb, sem_gb)

    def pair(i, carry):
        ca = 2 * i
        cb = 2 * i + 1

        @pl.when(i > 0)
        def _():
            wait_writeback(out_va, sem_oa)

        wait_gathers(tok_va, cmb_va, sem_ga)
        add_chunk(tok_va, cmb_va, out_va)
        fire_writeback(ca, out_va, sem_oa)

        @pl.when(i < NPAIR - 1)
        def _():
            prep_and_fire(ca + 2, tok_idx_a, cmb_idx_a, tok_va, cmb_va, sem_ga)

        @pl.when(i > 0)
        def _():
            wait_writeback(out_vb, sem_ob)

        wait_gathers(tok_vb, cmb_vb, sem_gb)
        add_chunk(tok_vb, cmb_vb, out_vb)
        fire_writeback(cb, out_vb, sem_ob)

        @pl.when(i < NPAIR - 1)
        def _():
            prep_and_fire(cb + 2, tok_idx_b, cmb_idx_b, tok_vb, cmb_vb, sem_gb)

        return carry

    lax.fori_loop(0, NPAIR, pair, 0)
    wait_writeback(out_va, sem_oa)
    wait_writeback(out_vb, sem_ob)


def kernel(sequence, segment_label, tok_table, seg_table):
    comb = _build_comb(seg_table)
    seq = sequence.reshape(TOTAL)
    lab = segment_label.reshape(TOTAL)
    tok_lin = (
        tok_table.T.reshape(D, VOCAB // 2, 2)
        .transpose(1, 2, 0)
        .reshape(VOCAB, D)
    )
    out = _emb(seq, lab, tok_lin, comb)
    return out.reshape(B, S, D)


# speedup vs baseline: 1.0049x; 1.0049x over previous
import functools

import numpy as np
import jax
import jax.numpy as jnp
from jax import lax
from jax.experimental import pallas as pl
from jax.experimental.pallas import tpu as pltpu
from jax.experimental.pallas import tpu_sc as plsc

VOCAB = 1000000
D = 64
B = 1024
S = 200

NC = 2
NS = 16
NW = NC * NS
TOTAL = B * S
PER_W = TOTAL // NW
CHUNK = 320
NCHUNK = PER_W // CHUNK
NPAIR = NCHUNK // 2
SLICES = [(off, min(128, CHUNK - off)) for off in range(0, CHUNK, 128)]


def _make_pe():
    pos = np.arange(S, dtype=np.float32)[:, None]
    div = np.exp(np.arange(0, D, 2, dtype=np.float32) * -(np.log(10000.0) / D))
    pe = np.zeros((S, D), dtype=np.float32)
    pe[:, 0::2] = np.sin(pos * div)
    pe[:, 1::2] = np.cos(pos * div)
    return pe


_PE = _make_pe()


def _comb_body(pe_ref, seg_ref, out_ref):
    pe = pe_ref[...]
    for l in range(3):
        out_ref[:, l * D:(l + 1) * D] = pe + seg_ref[l, :][None, :]


def _build_comb(seg_table):
    comb2 = pl.pallas_call(
        _comb_body,
        out_shape=jax.ShapeDtypeStruct((S, 3 * D), jnp.float32),
    )(_PE, seg_table)
    return comb2.reshape(3 * S, D)


_mesh = plsc.VectorSubcoreMesh(core_axis_name="c", subcore_axis_name="s")


@functools.partial(
    pl.kernel,
    mesh=_mesh,
    out_type=jax.ShapeDtypeStruct((TOTAL, D), jnp.float32),
    scratch_types=[
        pltpu.VMEM((CHUNK,), jnp.int32),
        pltpu.VMEM((CHUNK,), jnp.int32),
        pltpu.VMEM((CHUNK, D), jnp.float32),
        pltpu.VMEM((CHUNK, D), jnp.float32),
        pltpu.VMEM((CHUNK, D), jnp.float32),
        pltpu.VMEM((CHUNK,), jnp.int32),
        pltpu.VMEM((CHUNK,), jnp.int32),
        pltpu.VMEM((CHUNK, D), jnp.float32),
        pltpu.VMEM((CHUNK, D), jnp.float32),
        pltpu.VMEM((CHUNK, D), jnp.float32),
        pltpu.SemaphoreType.DMA,
        pltpu.SemaphoreType.DMA,
        pltpu.SemaphoreType.DMA,
        pltpu.SemaphoreType.DMA,
    ],
    compiler_params=pltpu.CompilerParams(use_tc_tiling_on_sc=False),
)
def _emb(seq_hbm, lab_hbm, tok_hbm, comb_hbm, out_hbm,
         tok_idx_a, cmb_idx_a, tok_va, cmb_va, out_va,
         tok_idx_b, cmb_idx_b, tok_vb, cmb_vb, out_vb,
         sem_ga, sem_gb, sem_oa, sem_ob):
    wid = lax.axis_index("s") * NC + lax.axis_index("c")
    w0 = wid * PER_W

    def prep_and_fire(c, tok_idx, cmb_idx, tok_v, cmb_v, sem_g):
        base = pl.multiple_of(w0 + c * CHUNK, CHUNK)
        pltpu.sync_copy(seq_hbm.at[pl.ds(base, CHUNK)], tok_idx)
        pltpu.sync_copy(lab_hbm.at[pl.ds(base, CHUNK)], cmb_idx)
        for k in range(CHUNK // 16):
            sl = pl.ds(k * 16, 16)
            lab = cmb_idx[sl]
            v = base + k * 16 + lax.iota(jnp.int32, 16)
            cmb_idx[sl] = lax.rem(v, S) * 3 + lab
        for off, ln in SLICES:
            pltpu.async_copy(tok_hbm.at[tok_idx.at[pl.ds(off, ln)]],
                             tok_v.at[pl.ds(off, ln)], sem_g)
            pltpu.async_copy(comb_hbm.at[cmb_idx.at[pl.ds(off, ln)]],
                             cmb_v.at[pl.ds(off, ln)], sem_g)

    def wait_gathers(tok_v, cmb_v, sem_g):
        pltpu.make_async_copy(tok_hbm.at[pl.ds(0, CHUNK)], tok_v, sem_g).wait()
        pltpu.make_async_copy(tok_hbm.at[pl.ds(0, CHUNK)], cmb_v, sem_g).wait()

    def add_chunk(tok_v, cmb_v, out_v):
        def add_rows(r, carry2):
            for j in range(2):
                i = r * 2 + j
                for k in range(D // 16):
                    sl = pl.ds(k * 16, 16)
                    out_v[i, sl] = tok_v[i, sl] + cmb_v[i, sl]
            return carry2

        lax.fori_loop(0, CHUNK // 2, add_rows, 0)

    def fire_writeback(c, out_v, sem_o):
        base = pl.multiple_of(w0 + c * CHUNK, CHUNK)
        pltpu.async_copy(out_v, out_hbm.at[pl.ds(base, CHUNK)], sem_o)

    def wait_writeback(out_v, sem_o):
        pltpu.make_async_copy(out_v, out_hbm.at[pl.ds(0, CHUNK)], sem_o).wait()

    prep_and_fire(0, tok_idx_a, cmb_idx_a, tok_va, cmb_va, sem_ga)
    prep_and_fire(1, tok_idx_b, cmb_idx_b, tok_vb, cmb_vb, sem_gb)

    def pair(i, carry):
        ca = 2 * i
        cb = 2 * i + 1

        @pl.when(i > 0)
        def _():
            wait_writeback(out_va, sem_oa)

        wait_gathers(tok_va, cmb_va, sem_ga)
        add_chunk(tok_va, cmb_va, out_va)
        fire_writeback(ca, out_va, sem_oa)

        @pl.when(i < NPAIR - 1)
        def _():
            prep_and_fire(ca + 2, tok_idx_a, cmb_idx_a, tok_va, cmb_va, sem_ga)

        @pl.when(i > 0)
        def _():
            wait_writeback(out_vb, sem_ob)

        wait_gathers(tok_vb, cmb_vb, sem_gb)
        add_chunk(tok_vb, cmb_vb, out_vb)
        fire_writeback(cb, out_vb, sem_ob)

        @pl.when(i < NPAIR - 1)
        def _():
            prep_and_fire(cb + 2, tok_idx_b, cmb_idx_b, tok_vb, cmb_vb, sem_gb)

        return carry

    lax.fori_loop(0, NPAIR, pair, 0)
    wait_writeback(out_va, sem_oa)
    wait_writeback(out_vb, sem_ob)


def kernel(sequence, segment_label, tok_table, seg_table):
    comb = _build_comb(seg_table)
    seq = sequence.reshape(TOTAL)
    lab = segment_label.reshape(TOTAL)
    out = _emb(seq, lab, tok_table, comb)
    return out.reshape(B, S, D)
